# baseline (speedup 1.0000x reference)
import functools

import jax
import jax.numpy as jnp
from jax import lax
from jax.experimental import pallas as pl
from jax.experimental.pallas import tpu as pltpu
from jax.experimental.pallas import tpu_sc as plsc

_CHUNK = 128


def _sc_gather(emb, idx3d, n_rows, d):
    nc = idx3d.shape[0] // 16
    chunks_per_w = idx3d.shape[1]

    mesh = plsc.VectorSubcoreMesh(core_axis_name="c", subcore_axis_name="s")

    @functools.partial(
        pl.kernel,
        mesh=mesh,
        out_type=jax.ShapeDtypeStruct((n_rows, d), jnp.float32),
        scratch_types=[
            pltpu.VMEM((chunks_per_w, _CHUNK), jnp.int32),
            pltpu.VMEM((_CHUNK, d), jnp.float32),
            pltpu.SemaphoreType.DMA,
        ],
        compiler_params=pltpu.CompilerParams(use_tc_tiling_on_sc=False),
    )
    def k(emb_hbm, idx_hbm, out_hbm, idx_v, rows_v, sem):
        wid = lax.axis_index("s") * nc + lax.axis_index("c")
        pltpu.sync_copy(idx_hbm.at[wid], idx_v)
        base = wid * (chunks_per_w * _CHUNK)

        def body(j, carry):
            pltpu.async_copy(emb_hbm.at[idx_v.at[j]], rows_v, sem).wait()
            pltpu.sync_copy(rows_v, out_hbm.at[pl.ds(base + j * _CHUNK, _CHUNK)])
            return carry

        lax.fori_loop(0, chunks_per_w, body, 0)

    return k(emb, idx3d)


_GW = 384


def _tc_block_body(A_ref, h0_ref, par_ref, al_ref, Wc_ref, cc_ref, bih_ref, out_ref, Ts_ref, Tg_ref, *, bb, L, S, D):
    bf = jnp.bfloat16
    cc = cc_ref[...]
    bih = bih_ref[...]

    def h0sel(b):
        hw = h0_ref[b]
        p = par_ref[b]
        return jnp.where(p == 1, hw[:, D : 2 * D], hw[:, 0:D])
    def stage1(b):
        h0b = h0sel(b).astype(bf)
        t0 = jnp.dot(h0b, Wc_ref[:, 0:_GW], preferred_element_type=jnp.float32) + cc[:, 0:_GW]
        Ts_ref[b, 0:L, :] = t0.astype(bf)
        t1 = jnp.dot(h0b, Wc_ref[:, _GW : 2 * _GW], preferred_element_type=jnp.float32) + cc[:, _GW : 2 * _GW]
        Ts_ref[b, L : 2 * L, :] = t1.astype(bf)
        t2 = jnp.dot(h0b, Wc_ref[:, 2 * _GW : 3 * _GW], preferred_element_type=jnp.float32) + cc[:, 2 * _GW : 3 * _GW]
        Tg_ref[b, 0:L, :] = t2

    def stage2(b):
        ab = A_ref[b].astype(bf)
        gi_r = jnp.dot(ab, Ts_ref[b, 0 : 2 * L, 0:128], preferred_element_type=jnp.float32)
        gi_z = jnp.dot(ab, Ts_ref[b, 0 : 2 * L, 128:256], preferred_element_type=jnp.float32)
        gi_n = jnp.dot(ab, Ts_ref[b, 0 : 2 * L, 256:384], preferred_element_type=jnp.float32)
        gh_r = Tg_ref[b, 0:L, 0:D]
        gh_z = Tg_ref[b, 0:L, 128 : 128 + D]
        gh_n = Tg_ref[b, 0:L, 256 : 256 + D]
        r = jax.nn.sigmoid(gi_r[:, 0:D] + bih[:, 0:D] + gh_r)
        z = jax.nn.sigmoid(gi_z[:, 0:D] + bih[:, 128 : 128 + D] + gh_z)
        n = jnp.tanh(gi_n[:, 0:D] + bih[:, 256 : 256 + D] + r * gh_n)
        h = (1.0 - z) * n + z * h0sel(b)
        al = al_ref[b]
        oh = (al == lax.broadcasted_iota(jnp.int32, (S, L), 1)).astype(bf)
        out_ref[b] = jnp.dot(oh, h.astype(bf), preferred_element_type=jnp.float32)

    stage1(0)
    for b in range(1, bb):
        stage1(b)
        stage2(b - 1)
    stage2(bb - 1)


def kernel(A, items, seq_alias, emb, W_in, b_in, W_out, b_out, W_ih, b_ih, W_hh, b_hh):
    B, L, twoL = A.shape
    S = seq_alias.shape[1]
    D = emb.shape[1]
    n_rows = B * L

    def gap3(M):
        z = jnp.zeros((M.shape[0], _GW), jnp.float32)
        return z.at[:, 0:64].set(M[:, 0:64]).at[:, 128:192].set(M[:, 64:128]).at[:, 256:320].set(M[:, 128:192])

    Wt, Wb = W_ih[:D, :], W_ih[D:, :]
    M_in, c_in = W_in @ Wt, b_in @ Wt
    M_out, c_out = W_out @ Wb, b_out @ Wb
    Wcat = jnp.concatenate([gap3(M_in), gap3(M_out), gap3(W_hh)], axis=1)
    ccat = jnp.concatenate([gap3(c_in[None]), gap3(c_out[None]), gap3(b_hh[None])], axis=1)
    bih2 = gap3(b_ih[None])

    V = emb.shape[0] - 1
    embw = emb[:V].reshape(V // 2, 2 * D)
    info = plsc.get_sparse_core_info()
    nw = info.num_cores * info.num_subcores
    it32 = items.astype(jnp.int32)
    idx3d = (it32 >> 1).reshape(nw, n_rows // (nw * _CHUNK), _CHUNK)
    h0w = _sc_gather(embw, idx3d, n_rows, 2 * D)
    h0_3d = h0w.reshape(B, L, 2 * D)
    par3 = (it32 & 1).reshape(B, L, 1)

    bb = 16
    alias3 = seq_alias.reshape(B, S, 1).astype(jnp.int32)
    body = functools.partial(_tc_block_body, bb=bb, L=L, S=S, D=D)
    h_seqs = pl.pallas_call(
        body,
        grid=(B // bb,),
        in_specs=[
            pl.BlockSpec((bb, L, twoL), lambda i: (i, 0, 0)),
            pl.BlockSpec((bb, L, 2 * D), lambda i: (i, 0, 0)),
            pl.BlockSpec((bb, L, 1), lambda i: (i, 0, 0)),
            pl.BlockSpec((bb, S, 1), lambda i: (i, 0, 0)),
            pl.BlockSpec((D, 3 * _GW), lambda i: (0, 0)),
            pl.BlockSpec((1, 3 * _GW), lambda i: (0, 0)),
            pl.BlockSpec((1, _GW), lambda i: (0, 0)),
        ],
        out_specs=pl.BlockSpec((bb, S, D), lambda i: (i, 0, 0)),
        out_shape=jax.ShapeDtypeStruct((B, S, D), jnp.float32),
        scratch_shapes=[
            pltpu.VMEM((bb, 2 * L + 4, _GW), jnp.bfloat16),
            pltpu.VMEM((bb, L + 6, _GW), jnp.float32),
        ],
        compiler_params=pltpu.CompilerParams(dimension_semantics=("parallel",)),
    )(A, h0_3d, par3, alias3, Wcat.astype(jnp.bfloat16), ccat, bih2)
    return h_seqs


# speedup vs baseline: 1.0893x; 1.0893x over previous
import functools

import jax
import jax.numpy as jnp
from jax import lax
from jax.experimental import pallas as pl
from jax.experimental.pallas import tpu as pltpu
from jax.experimental.pallas import tpu_sc as plsc

_CHUNK = 128


def _sc_gather(emb, idx3d, n_rows, d):
    nc = idx3d.shape[0] // 16
    chunks_per_w = idx3d.shape[1]

    mesh = plsc.VectorSubcoreMesh(core_axis_name="c", subcore_axis_name="s")

    @functools.partial(
        pl.kernel,
        mesh=mesh,
        out_type=jax.ShapeDtypeStruct((n_rows, d), jnp.float32),
        scratch_types=[
            pltpu.VMEM((chunks_per_w, _CHUNK), jnp.int32),
            pltpu.VMEM((_CHUNK, d), jnp.float32),
            pltpu.SemaphoreType.DMA,
        ],
        compiler_params=pltpu.CompilerParams(use_tc_tiling_on_sc=False),
    )
    def k(emb_hbm, idx_hbm, out_hbm, idx_v, rows_v, sem):
        wid = lax.axis_index("s") * nc + lax.axis_index("c")
        pltpu.sync_copy(idx_hbm.at[wid], idx_v)
        base = wid * (chunks_per_w * _CHUNK)

        def body(j, carry):
            pltpu.async_copy(emb_hbm.at[idx_v.at[j]], rows_v, sem).wait()
            pltpu.sync_copy(rows_v, out_hbm.at[pl.ds(base + j * _CHUNK, _CHUNK)])
            return carry

        lax.fori_loop(0, chunks_per_w, body, 0)

    return k(emb, idx3d)


_GW = 384


def _tc_block_body(A_ref, h0_ref, al_ref, Wc_ref, bihn_ref, out_ref, Ts_ref, Tg_ref, *, bb, L, S, D):
    bf = jnp.bfloat16
    bihn = bihn_ref[...]
    ones = jnp.ones((L, D), bf)
    def stage1(b):
        h0aug = jnp.concatenate([h0_ref[b].astype(bf), ones], axis=1)
        t0 = jnp.dot(h0aug, Wc_ref[:, 0:_GW], preferred_element_type=jnp.float32)
        Ts_ref[b, 0:L, :] = t0.astype(bf)
        t1 = jnp.dot(h0aug, Wc_ref[:, _GW : 2 * _GW], preferred_element_type=jnp.float32)
        Ts_ref[b, L : 2 * L, :] = t1.astype(bf)
        t2 = jnp.dot(h0aug, Wc_ref[:, 2 * _GW : 3 * _GW], preferred_element_type=jnp.float32)
        Tg_ref[b, 0:L, :] = t2.astype(bf)

    def stage2(b):
        ab = A_ref[b].astype(bf)
        gi_rz = jnp.dot(ab, Ts_ref[b, 0 : 2 * L, 0:256], preferred_element_type=jnp.float32)
        gi_n = jnp.dot(ab, Ts_ref[b, 0 : 2 * L, 256:384], preferred_element_type=jnp.float32)
        gh_r = Tg_ref[b, 0:L, 0:D]
        gh_z = Tg_ref[b, 0:L, 128 : 128 + D]
        gh_n = Tg_ref[b, 0:L, 256 : 256 + D]
        r = jax.nn.sigmoid(gi_rz[:, 0:D] + gh_r)
        z = jax.nn.sigmoid(gi_rz[:, 128 : 128 + D] + gh_z)
        n = jnp.tanh(gi_n[:, 0:D] + bihn + r * gh_n)
        h0b = h0_ref[b]
        h = (1.0 - z) * n + z * h0b
        al = al_ref[b]
        oh = (al == lax.broadcasted_iota(jnp.int32, (S, L), 1)).astype(bf)
        out_ref[b] = jnp.dot(oh, h.astype(bf), preferred_element_type=jnp.float32)

    stage1(0)
    for b in range(1, bb):
        stage1(b)
        stage2(b - 1)
    stage2(bb - 1)


def kernel(A, items, seq_alias, emb, W_in, b_in, W_out, b_out, W_ih, b_ih, W_hh, b_hh):
    B, L, twoL = A.shape
    S = seq_alias.shape[1]
    D = emb.shape[1]
    n_rows = B * L

    def gap3(M):
        z = jnp.zeros((M.shape[0], _GW), jnp.float32)
        return z.at[:, 0:64].set(M[:, 0:64]).at[:, 128:192].set(M[:, 64:128]).at[:, 256:320].set(M[:, 128:192])

    Wt, Wb = W_ih[:D, :], W_ih[D:, :]
    M_in, c_in = W_in @ Wt, b_in @ Wt
    M_out, c_out = W_out @ Wb, b_out @ Wb
    bih_rz = gap3(b_ih[None]).at[:, 256:320].set(0.0)
    Wcat = jnp.concatenate([gap3(M_in), gap3(M_out), gap3(W_hh)], axis=1)
    ccat = jnp.concatenate(
        [gap3(c_in[None]), gap3(c_out[None]), gap3(b_hh[None]) + bih_rz], axis=1
    )
    Wcat_aug = jnp.zeros((2 * D, 3 * _GW), jnp.float32)
    Wcat_aug = Wcat_aug.at[0:D, :].set(Wcat).at[D, :].set(ccat[0])
    bihn = b_ih[None, 2 * D : 3 * D]

    info = plsc.get_sparse_core_info()
    nw = info.num_cores * info.num_subcores
    idx3d = items.reshape(nw, n_rows // (nw * _CHUNK), _CHUNK).astype(jnp.int32)
    h0 = _sc_gather(emb, idx3d, n_rows, D)
    h0_3d = h0.reshape(B, L, D)

    bb = 16
    alias3 = seq_alias.reshape(B, S, 1).astype(jnp.int32)
    body = functools.partial(_tc_block_body, bb=bb, L=L, S=S, D=D)
    h_seqs = pl.pallas_call(
        body,
        grid=(B // bb,),
        in_specs=[
            pl.BlockSpec((bb, L, twoL), lambda i: (i, 0, 0)),
            pl.BlockSpec((bb, L, D), lambda i: (i, 0, 0)),
            pl.BlockSpec((bb, S, 1), lambda i: (i, 0, 0)),
            pl.BlockSpec((2 * D, 3 * _GW), lambda i: (0, 0)),
            pl.BlockSpec((1, D), lambda i: (0, 0)),
        ],
        out_specs=pl.BlockSpec((bb, S, D), lambda i: (i, 0, 0)),
        out_shape=jax.ShapeDtypeStruct((B, S, D), jnp.float32),
        scratch_shapes=[
            pltpu.VMEM((bb, 2 * L + 4, _GW), jnp.bfloat16),
            pltpu.VMEM((bb, L + 6, _GW), jnp.bfloat16),
        ],
        compiler_params=pltpu.CompilerParams(dimension_semantics=("parallel",)),
    )(A, h0_3d, alias3, Wcat_aug.astype(jnp.bfloat16), bihn)
    return h_seqs


# speedup vs baseline: 1.1063x; 1.0156x over previous
import functools

import jax
import jax.numpy as jnp
from jax import lax
from jax.experimental import pallas as pl
from jax.experimental.pallas import tpu as pltpu
from jax.experimental.pallas import tpu_sc as plsc

_CHUNK = 128


def _sc_gather(emb, idx3d, n_rows, d):
    nc = idx3d.shape[0] // 16
    chunks_per_w = idx3d.shape[1]

    mesh = plsc.VectorSubcoreMesh(core_axis_name="c", subcore_axis_name="s")

    @functools.partial(
        pl.kernel,
        mesh=mesh,
        out_type=jax.ShapeDtypeStruct((n_rows, d), jnp.float32),
        scratch_types=[
            pltpu.VMEM((chunks_per_w, _CHUNK), jnp.int32),
            pltpu.VMEM((_CHUNK, d), jnp.float32),
            pltpu.SemaphoreType.DMA,
        ],
        compiler_params=pltpu.CompilerParams(use_tc_tiling_on_sc=False),
    )
    def k(emb_hbm, idx_hbm, out_hbm, idx_v, rows_v, sem):
        wid = lax.axis_index("s") * nc + lax.axis_index("c")
        pltpu.sync_copy(idx_hbm.at[wid], idx_v)
        base = wid * (chunks_per_w * _CHUNK)

        def body(j, carry):
            pltpu.async_copy(emb_hbm.at[idx_v.at[j]], rows_v, sem).wait()
            pltpu.sync_copy(rows_v, out_hbm.at[pl.ds(base + j * _CHUNK, _CHUNK)])
            return carry

        lax.fori_loop(0, chunks_per_w, body, 0)

    return k(emb, idx3d)


_GW = 384


def _tc_block_body(A_ref, h0_ref, al_ref, Wc_ref, bihn_ref, out_ref, Ts_ref, Tg_ref, *, bb, L, S, D):
    bf = jnp.bfloat16
    bihn = bihn_ref[...]
    ones = jnp.ones((L, D), bf)
    def stage1(b):
        h0aug = jnp.concatenate([h0_ref[b].astype(bf), ones], axis=1)
        t0 = jnp.dot(h0aug, Wc_ref[:, 0:_GW], preferred_element_type=jnp.float32)
        Ts_ref[b, 0:L, :] = t0.astype(bf)
        t1 = jnp.dot(h0aug, Wc_ref[:, _GW : 2 * _GW], preferred_element_type=jnp.float32)
        Ts_ref[b, L : 2 * L, :] = t1.astype(bf)
        t2 = jnp.dot(h0aug, Wc_ref[:, 2 * _GW : 3 * _GW], preferred_element_type=jnp.float32)
        Tg_ref[b, 0:L, :] = t2.astype(bf)

    def stage2(b):
        ab = A_ref[b].astype(bf)
        gi_rz = jnp.dot(ab, Ts_ref[b, 0 : 2 * L, 0:256], preferred_element_type=jnp.float32)
        gi_n = jnp.dot(ab, Ts_ref[b, 0 : 2 * L, 256:384], preferred_element_type=jnp.float32)
        gh_r = Tg_ref[b, 0:L, 0:D]
        gh_z = Tg_ref[b, 0:L, 128 : 128 + D]
        gh_n = Tg_ref[b, 0:L, 256 : 256 + D]
        r = jax.nn.sigmoid(gi_rz[:, 0:D] + gh_r)
        z = jax.nn.sigmoid(gi_rz[:, 128 : 128 + D] + gh_z)
        n = jnp.tanh(gi_n[:, 0:D] + bihn + r * gh_n)
        h0b = h0_ref[b]
        h = (1.0 - z) * n + z * h0b
        al = al_ref[b]
        oh = (al == lax.broadcasted_iota(jnp.int32, (S, L), 1)).astype(bf)
        out_ref[b] = jnp.dot(oh, h.astype(bf), preferred_element_type=jnp.float32)

    stage1(0)
    for b in range(1, bb):
        stage1(b)
        stage2(b - 1)
    stage2(bb - 1)


def kernel(A, items, seq_alias, emb, W_in, b_in, W_out, b_out, W_ih, b_ih, W_hh, b_hh):
    B, L, twoL = A.shape
    S = seq_alias.shape[1]
    D = emb.shape[1]
    n_rows = B * L

    def gap3(M):
        z = jnp.zeros((M.shape[0], _GW), jnp.float32)
        return z.at[:, 0:64].set(M[:, 0:64]).at[:, 128:192].set(M[:, 64:128]).at[:, 256:320].set(M[:, 128:192])

    Wt, Wb = W_ih[:D, :], W_ih[D:, :]
    M_in, c_in = W_in @ Wt, b_in @ Wt
    M_out, c_out = W_out @ Wb, b_out @ Wb
    bih_rz = gap3(b_ih[None]).at[:, 256:320].set(0.0)
    Wcat = jnp.concatenate([gap3(M_in), gap3(M_out), gap3(W_hh)], axis=1)
    ccat = jnp.concatenate(
        [gap3(c_in[None]), gap3(c_out[None]), gap3(b_hh[None]) + bih_rz], axis=1
    )
    Wcat_aug = jnp.zeros((2 * D, 3 * _GW), jnp.float32)
    Wcat_aug = Wcat_aug.at[0:D, :].set(Wcat).at[D, :].set(ccat[0])
    bihn = b_ih[None, 2 * D : 3 * D]

    info = plsc.get_sparse_core_info()
    nw = info.num_cores * info.num_subcores
    idx3d = items.reshape(nw, n_rows // (nw * _CHUNK), _CHUNK).astype(jnp.int32)
    h0 = _sc_gather(emb, idx3d, n_rows, D)
    h0_3d = h0.reshape(B, L, D)

    bb = 32
    alias3 = seq_alias.reshape(B, S, 1).astype(jnp.int32)
    body = functools.partial(_tc_block_body, bb=bb, L=L, S=S, D=D)
    h_seqs = pl.pallas_call(
        body,
        grid=(B // bb,),
        in_specs=[
            pl.BlockSpec((bb, L, twoL), lambda i: (i, 0, 0)),
            pl.BlockSpec((bb, L, D), lambda i: (i, 0, 0)),
            pl.BlockSpec((bb, S, 1), lambda i: (i, 0, 0)),
            pl.BlockSpec((2 * D, 3 * _GW), lambda i: (0, 0)),
            pl.BlockSpec((1, D), lambda i: (0, 0)),
        ],
        out_specs=pl.BlockSpec((bb, S, D), lambda i: (i, 0, 0)),
        out_shape=jax.ShapeDtypeStruct((B, S, D), jnp.float32),
        scratch_shapes=[
            pltpu.VMEM((bb, 2 * L + 4, _GW), jnp.bfloat16),
            pltpu.VMEM((bb, L + 6, _GW), jnp.bfloat16),
        ],
        compiler_params=pltpu.CompilerParams(dimension_semantics=("parallel",)),
    )(A, h0_3d, alias3, Wcat_aug.astype(jnp.bfloat16), bihn)
    return h_seqs


# speedup vs baseline: 1.1154x; 1.0082x over previous
import functools

import jax
import jax.numpy as jnp
from jax import lax
from jax.experimental import pallas as pl
from jax.experimental.pallas import tpu as pltpu
from jax.experimental.pallas import tpu_sc as plsc

_CHUNK = 128


def _sc_gather(emb, idx3d, n_rows, d):
    nc = idx3d.shape[0] // 16
    chunks_per_w = idx3d.shape[1]

    mesh = plsc.VectorSubcoreMesh(core_axis_name="c", subcore_axis_name="s")

    @functools.partial(
        pl.kernel,
        mesh=mesh,
        out_type=jax.ShapeDtypeStruct((n_rows, d), jnp.float32),
        scratch_types=[
            pltpu.VMEM((chunks_per_w, _CHUNK), jnp.int32),
            pltpu.VMEM((_CHUNK, d), jnp.float32),
            pltpu.SemaphoreType.DMA,
        ],
        compiler_params=pltpu.CompilerParams(use_tc_tiling_on_sc=False),
    )
    def k(emb_hbm, idx_hbm, out_hbm, idx_v, rows_v, sem):
        wid = lax.axis_index("s") * nc + lax.axis_index("c")
        pltpu.sync_copy(idx_hbm.at[wid], idx_v)
        base = wid * (chunks_per_w * _CHUNK)

        def body(j, carry):
            pltpu.async_copy(emb_hbm.at[idx_v.at[j]], rows_v, sem).wait()
            pltpu.sync_copy(rows_v, out_hbm.at[pl.ds(base + j * _CHUNK, _CHUNK)])
            return carry

        lax.fori_loop(0, chunks_per_w, body, 0)

    return k(emb, idx3d)


_GW = 384


def _tc_block_body(A_ref, h0_ref, al_ref, Wc_ref, bihn_ref, out_ref, Ts_ref, Tg_ref, *, bb, L, S, D):
    bf = jnp.bfloat16
    bihn = bihn_ref[...]
    ones = jnp.ones((L, D), bf)
    def stage1(b):
        h0aug = jnp.concatenate([h0_ref[b].astype(bf), ones], axis=1)
        t0 = jnp.dot(h0aug, Wc_ref[:, 0:_GW], preferred_element_type=jnp.float32)
        Ts_ref[b, 0:L, :] = t0.astype(bf)
        t1 = jnp.dot(h0aug, Wc_ref[:, _GW : 2 * _GW], preferred_element_type=jnp.float32)
        Ts_ref[b, L : 2 * L, :] = t1.astype(bf)
        t2 = jnp.dot(h0aug, Wc_ref[:, 2 * _GW : 3 * _GW], preferred_element_type=jnp.float32)
        Tg_ref[b, 0:L, :] = t2.astype(bf)

    def stage2(b):
        ab = A_ref[b].astype(bf)
        gi_rz = jnp.dot(ab, Ts_ref[b, 0 : 2 * L, 0:256], preferred_element_type=jnp.float32)
        gi_n = jnp.dot(ab, Ts_ref[b, 0 : 2 * L, 256:384], preferred_element_type=jnp.float32)
        gh_r = Tg_ref[b, 0:L, 0:D]
        gh_z = Tg_ref[b, 0:L, 128 : 128 + D]
        gh_n = Tg_ref[b, 0:L, 256 : 256 + D]
        r = jax.nn.sigmoid(gi_rz[:, 0:D] + gh_r)
        z = jax.nn.sigmoid(gi_rz[:, 128 : 128 + D] + gh_z)
        n = jnp.tanh(gi_n[:, 0:D] + bihn + r * gh_n)
        h0b = h0_ref[b]
        h = (1.0 - z) * n + z * h0b
        al = al_ref[b]
        oh = (al == lax.broadcasted_iota(jnp.int32, (S, L), 1)).astype(bf)
        out_ref[b] = jnp.dot(oh, h.astype(bf), preferred_element_type=jnp.float32)

    stage1(0)
    for b in range(1, bb):
        stage1(b)
        stage2(b - 1)
    stage2(bb - 1)


def kernel(A, items, seq_alias, emb, W_in, b_in, W_out, b_out, W_ih, b_ih, W_hh, b_hh):
    B, L, twoL = A.shape
    S = seq_alias.shape[1]
    D = emb.shape[1]
    n_rows = B * L

    def gap3(M):
        z = jnp.zeros((M.shape[0], _GW), jnp.float32)
        return z.at[:, 0:64].set(M[:, 0:64]).at[:, 128:192].set(M[:, 64:128]).at[:, 256:320].set(M[:, 128:192])

    Wt, Wb = W_ih[:D, :], W_ih[D:, :]
    M_in, c_in = W_in @ Wt, b_in @ Wt
    M_out, c_out = W_out @ Wb, b_out @ Wb
    bih_rz = gap3(b_ih[None]).at[:, 256:320].set(0.0)
    Wcat = jnp.concatenate([gap3(M_in), gap3(M_out), gap3(W_hh)], axis=1)
    ccat = jnp.concatenate(
        [gap3(c_in[None]), gap3(c_out[None]), gap3(b_hh[None]) + bih_rz], axis=1
    )
    Wcat_aug = jnp.zeros((2 * D, 3 * _GW), jnp.float32)
    Wcat_aug = Wcat_aug.at[0:D, :].set(Wcat).at[D, :].set(ccat[0])
    bihn = b_ih[None, 2 * D : 3 * D]

    info = plsc.get_sparse_core_info()
    nw = info.num_cores * info.num_subcores
    idx3d = items.reshape(nw, n_rows // (nw * _CHUNK), _CHUNK).astype(jnp.int32)
    h0 = _sc_gather(emb, idx3d, n_rows, D)
    h0_3d = h0.reshape(B, L, D)

    bb = 64
    alias3 = seq_alias.reshape(B, S, 1).astype(jnp.int32)
    body = functools.partial(_tc_block_body, bb=bb, L=L, S=S, D=D)
    h_seqs = pl.pallas_call(
        body,
        grid=(B // bb,),
        in_specs=[
            pl.BlockSpec((bb, L, twoL), lambda i: (i, 0, 0)),
            pl.BlockSpec((bb, L, D), lambda i: (i, 0, 0)),
            pl.BlockSpec((bb, S, 1), lambda i: (i, 0, 0)),
            pl.BlockSpec((2 * D, 3 * _GW), lambda i: (0, 0)),
            pl.BlockSpec((1, D), lambda i: (0, 0)),
        ],
        out_specs=pl.BlockSpec((bb, S, D), lambda i: (i, 0, 0)),
        out_shape=jax.ShapeDtypeStruct((B, S, D), jnp.float32),
        scratch_shapes=[
            pltpu.VMEM((bb, 2 * L + 4, _GW), jnp.bfloat16),
            pltpu.VMEM((bb, L + 6, _GW), jnp.bfloat16),
        ],
        compiler_params=pltpu.CompilerParams(dimension_semantics=("parallel",)),
    )(A, h0_3d, alias3, Wcat_aug.astype(jnp.bfloat16), bihn)
    return h_seqs
